# baseline (speedup 1.0000x reference)
import functools

import jax
import jax.numpy as jnp
from jax import lax
from jax.experimental import pallas as pl
from jax.experimental.pallas import tpu as pltpu
from jax.experimental.pallas import tpu_sc as plsc

N_USER = 1000000
N_ITEM = 100000
N_INP = 64
BATCH = 16384

_info = plsc.get_sparse_core_info()
_NC, _NS = _info.num_cores, _info.num_subcores
_NW = _NC * _NS
_BPW = BATCH // _NW


def _gather_body(user_hbm, item_hbm, nid_u_hbm, nid_i_hbm,
                 out_u_hbm, out_i_hbm,
                 idx_u, idx_i, rows_u, rows_i, sem_u, sem_i):
    wid = lax.axis_index("s") * _NC + lax.axis_index("c")
    base = wid * _BPW
    pltpu.sync_copy(nid_u_hbm.at[pl.ds(base, _BPW)], idx_u)
    pltpu.sync_copy(nid_i_hbm.at[pl.ds(base, _BPW)], idx_i)
    cp_u = pltpu.async_copy(user_hbm.at[idx_u], rows_u, sem_u)
    cp_i = pltpu.async_copy(item_hbm.at[idx_i], rows_i, sem_i)
    cp_u.wait()
    pltpu.sync_copy(rows_u, out_u_hbm.at[pl.ds(base, _BPW)])
    cp_i.wait()
    pltpu.sync_copy(rows_i, out_i_hbm.at[pl.ds(base, _BPW)])


@jax.jit
def kernel(embed_user, embed_item, nid_user, nid_item):
    mesh = plsc.VectorSubcoreMesh(core_axis_name="c", subcore_axis_name="s")
    run = functools.partial(
        pl.kernel,
        mesh=mesh,
        out_type=(
            jax.ShapeDtypeStruct((BATCH, N_INP), jnp.float32),
            jax.ShapeDtypeStruct((BATCH, N_INP), jnp.float32),
        ),
        scratch_types=[
            pltpu.VMEM((_BPW,), jnp.int32),
            pltpu.VMEM((_BPW,), jnp.int32),
            pltpu.VMEM((_BPW, N_INP), jnp.float32),
            pltpu.VMEM((_BPW, N_INP), jnp.float32),
            pltpu.SemaphoreType.DMA,
            pltpu.SemaphoreType.DMA,
        ],
        compiler_params=pltpu.CompilerParams(use_tc_tiling_on_sc=False),
    )(_gather_body)
    return run(embed_user, embed_item, nid_user, nid_item)


# speedup vs baseline: 2.4155x; 2.4155x over previous
import functools

import jax
import jax.numpy as jnp
from jax import lax
from jax.experimental import pallas as pl
from jax.experimental.pallas import tpu as pltpu
from jax.experimental.pallas import tpu_sc as plsc

N_USER = 1000000
N_ITEM = 100000
N_INP = 64
BATCH = 16384

_info = plsc.get_sparse_core_info()
_NC, _NS = _info.num_cores, _info.num_subcores
_NW = _NC * _NS
_BPW = BATCH // _NW
_C = 256
_CI = 128


def _gather_body(user3, item3, nid_u_hbm, nid_i_hbm,
                 out_u_hbm, out_i_hbm,
                 idx_u, idx_i,
                 rows_u0, rows_u1, rows_i0, rows_i1,
                 sem_u0, sem_u1, sem_i0, sem_i1):
    wid = lax.axis_index("s") * _NC + lax.axis_index("c")
    base = wid * _BPW
    pltpu.sync_copy(nid_u_hbm.at[pl.ds(base, _BPW)], idx_u)
    pltpu.sync_copy(nid_i_hbm.at[pl.ds(base, _BPW)], idx_i)

    def burst(tab, idx, rows, sem, off, cnt):
        def issue(k, c):
            vec = idx[pl.ds(off + k * 16, 16)]
            for i in range(16):
                v = vec[i]
                pltpu.make_async_copy(tab.at[v >> 3, v & 7],
                                      rows.at[k * 16 + i], sem).start()
            return c
        lax.fori_loop(0, cnt // 16, issue, 0)

    def drain_write(rows, sem, out, off, cnt):
        pltpu.make_async_copy(out.at[pl.ds(base + off, cnt)], rows, sem).wait()
        pltpu.sync_copy(rows, out.at[pl.ds(base + off, cnt)])

    burst(user3, idx_u, rows_u0, sem_u0, 0, _C)
    burst(user3, idx_u, rows_u1, sem_u1, _C, _C)
    burst(item3, idx_i, rows_i0, sem_i0, 0, _CI)
    burst(item3, idx_i, rows_i1, sem_i1, _CI, _CI)
    drain_write(rows_i0, sem_i0, out_i_hbm, 0, _CI)
    burst(item3, idx_i, rows_i0, sem_i0, 2 * _CI, _CI)
    drain_write(rows_i1, sem_i1, out_i_hbm, _CI, _CI)
    burst(item3, idx_i, rows_i1, sem_i1, 3 * _CI, _CI)
    drain_write(rows_u0, sem_u0, out_u_hbm, 0, _C)
    drain_write(rows_u1, sem_u1, out_u_hbm, _C, _C)
    drain_write(rows_i0, sem_i0, out_i_hbm, 2 * _CI, _CI)
    drain_write(rows_i1, sem_i1, out_i_hbm, 3 * _CI, _CI)


@jax.jit
def kernel(embed_user, embed_item, nid_user, nid_item):
    user3 = embed_user.reshape(N_USER // 8, 8, N_INP)
    item3 = embed_item.reshape(N_ITEM // 8, 8, N_INP)
    mesh = plsc.VectorSubcoreMesh(core_axis_name="c", subcore_axis_name="s")
    run = functools.partial(
        pl.kernel,
        mesh=mesh,
        out_type=(
            jax.ShapeDtypeStruct((BATCH, N_INP), jnp.float32),
            jax.ShapeDtypeStruct((BATCH, N_INP), jnp.float32),
        ),
        scratch_types=[
            pltpu.VMEM((_BPW,), jnp.int32),
            pltpu.VMEM((_BPW,), jnp.int32),
            pltpu.VMEM((_C, N_INP), jnp.float32),
            pltpu.VMEM((_C, N_INP), jnp.float32),
            pltpu.VMEM((_CI, N_INP), jnp.float32),
            pltpu.VMEM((_CI, N_INP), jnp.float32),
            pltpu.SemaphoreType.DMA,
            pltpu.SemaphoreType.DMA,
            pltpu.SemaphoreType.DMA,
            pltpu.SemaphoreType.DMA,
        ],
        compiler_params=pltpu.CompilerParams(needs_layout_passes=False),
    )(_gather_body)
    return run(user3, item3, nid_user, nid_item)
